# baseline (speedup 1.0000x reference)
import functools

import jax
import jax.numpy as jnp
from jax import lax
from jax.experimental import pallas as pl
from jax.experimental.pallas import tpu as pltpu
from jax.experimental.pallas import tpu_sc as plsc

VOCAB = 100000
EMBED = 64
BATCH = 4096
TP = 8

_info = plsc.get_sparse_core_info()
_NC, _NS, _L = _info.num_cores, _info.num_subcores, _info.num_lanes
_NW = _NC * _NS
_CHUNKS = BATCH // TP
_WPT = _NW // TP
_CPW = _CHUNKS // _WPT


def _sc_body(xp_hbm, t2_hbm, out_hbm, xbuf, pairidx, pairs, rows, sem, gsem):
    wid = lax.axis_index("s") * _NC + lax.axis_index("c")
    t = wid // _WPT
    b = wid % _WPT
    pltpu.sync_copy(xp_hbm.at[wid, 0], xbuf)
    for jj in range(_CPW // _L):
        pairidx[pl.ds(jj * _L, _L)] = lax.shift_right_logical(
            xbuf[pl.ds(jj * _L, _L)], 1)
    pltpu.async_copy(t2_hbm.at[pairidx], pairs, gsem).wait()
    for jj in range(_CPW // _L):
        vx = xbuf[pl.ds(jj * _L, _L)]
        for l in range(_L):
            j = jj * _L + l
            par = (vx[l] & 1) * EMBED
            for k in range(EMBED // _L):
                rows[j // 2, pl.ds((j % 2) * EMBED + k * _L, _L)] = (
                    pairs[j, pl.ds(par + k * _L, _L)])
    pltpu.sync_copy(rows, out_hbm.at[t, pl.ds(b * (_CPW // 2), _CPW // 2)])


_gather_embed = functools.partial(
    pl.kernel,
    out_type=jax.ShapeDtypeStruct((TP, _CHUNKS // 2, 2 * EMBED), jnp.float32),
    mesh=plsc.VectorSubcoreMesh(core_axis_name="c", subcore_axis_name="s"),
    scratch_types=[
        pltpu.VMEM((_CPW,), jnp.int32),
        pltpu.VMEM((_CPW,), jnp.int32),
        pltpu.VMEM((_CPW, 2 * EMBED), jnp.float32),
        pltpu.VMEM((_CPW // 2, 2 * EMBED), jnp.float32),
        pltpu.SemaphoreType.DMA,
        pltpu.SemaphoreType.DMA,
    ],
)(_sc_body)


@jax.jit
def kernel(x, table):
    xp = (jnp.asarray(x, jnp.int32)
          .reshape(_CHUNKS, TP).T.reshape(_NW, 1, _CPW))
    out = _gather_embed(xp, table.reshape(VOCAB // 2, 2 * EMBED))
    return out.reshape(TP, _CHUNKS * EMBED)


# speedup vs baseline: 1.2497x; 1.2497x over previous
import functools

import jax
import jax.numpy as jnp
from jax import lax
from jax.experimental import pallas as pl
from jax.experimental.pallas import tpu as pltpu
from jax.experimental.pallas import tpu_sc as plsc

VOCAB = 100000
EMBED = 64
BATCH = 4096
TP = 8

_info = plsc.get_sparse_core_info()
_NC, _NS, _L = _info.num_cores, _info.num_subcores, _info.num_lanes
_NW = _NC * _NS
_CHUNKS = BATCH // TP
_DPW = EMBED // _NW
_IB = BATCH // _L
_CK = 128


def _sc_body(xp_hbm, t1_hbm, out_hbm, xbuf, posbuf, rowbuf, sem):
    wid = lax.axis_index("s") * _NC + lax.axis_index("c")
    pltpu.sync_copy(xp_hbm, xbuf)
    copies = []
    for dd in range(_DPW):
        d = wid * _DPW + dd
        for i in range(_IB):
            sl = pl.ds(i * _L, _L)
            posbuf[pl.ds(dd * BATCH + i * _L, _L)] = xbuf[sl] + d * VOCAB
        for jj in range(BATCH // _CK):
            o = dd * BATCH + jj * _CK
            copies.append(pltpu.async_copy(
                t1_hbm.at[posbuf.at[pl.ds(o, _CK)]],
                rowbuf.at[pl.ds(o, _CK)], sem))
    for c in copies:
        c.wait()
    pltpu.sync_copy(rowbuf, out_hbm.at[wid])


_gather_embed = functools.partial(
    pl.kernel,
    out_type=jax.ShapeDtypeStruct((_NW, _DPW * BATCH), jnp.float32),
    mesh=plsc.VectorSubcoreMesh(core_axis_name="c", subcore_axis_name="s"),
    scratch_types=[
        pltpu.VMEM((BATCH,), jnp.int32),
        pltpu.VMEM((_DPW * BATCH,), jnp.int32),
        pltpu.VMEM((_DPW * BATCH,), jnp.float32),
        pltpu.SemaphoreType.DMA,
    ],
    compiler_params=pltpu.CompilerParams(use_tc_tiling_on_sc=False),
)(_sc_body)


@jax.jit
def kernel(x, table):
    xp = jnp.asarray(x, jnp.int32).reshape(_CHUNKS, TP).T.reshape(BATCH)
    out_d = _gather_embed(xp, table.T.reshape(-1))
    return (out_d.reshape(EMBED, TP, _CHUNKS)
            .transpose(1, 2, 0).reshape(TP, _CHUNKS * EMBED))
